# Initial kernel scaffold; baseline (speedup 1.0000x reference)
import functools

import jax
import jax.numpy as jnp
from jax import lax
from jax.experimental import pallas as pl
from jax.experimental.pallas import tpu as pltpu
from jax.experimental.pallas import tpu_sc as plsc

NQ = 10000
D = 256
NH = 8
DH = 32
P = 4
QQ = 2
H = 100
W = 100
HP = 102
WP = 104
NW = 32
CI = 4
IPW = 320
NIP = NW * IPW
NCHUNK = IPW // CI
RT = QQ * HP * WP * NH

_XSTEP = NH
_YSTEP = WP * NH
_QSTEP = HP * WP * NH


def _vproj_body(v_ref, w_ref, b_ref, o_ref):
    o_ref[...] = jnp.dot(v_ref[...], w_ref[...],
                         preferred_element_type=jnp.float32) + b_ref[...]


def _prep_body(v0_ref, q_ref, ref_ref, wxv, wxq, wyv, wyq, wav, waq,
               bx_ref, by_ref, ba_ref, g_ref, idx_ref, w_ref, *, rows):
    v0 = v0_ref[...]
    q = q_ref[...]
    dot = functools.partial(jnp.dot, preferred_element_type=jnp.float32)
    offx = dot(v0, wxv[...]) + dot(q, wxq[...]) + bx_ref[...]
    offy = dot(v0, wyv[...]) + dot(q, wyq[...]) + by_ref[...]
    logit = dot(v0, wav[...]) + dot(q, waq[...]) + ba_ref[...]
    logit = logit - jnp.max(logit, axis=1, keepdims=True)
    e = jnp.exp(logit)
    a = e / dot(e, g_ref[...])

    refb = ref_ref[...]
    px = refb[:, 0:1] * jnp.float32(W) + offx - jnp.float32(0.5)
    py = refb[:, 1:2] * jnp.float32(H) + offy - jnp.float32(0.5)
    x0 = jnp.floor(px)
    y0 = jnp.floor(py)
    fx = px - x0
    fy = py - y0
    xh = jnp.clip(x0, -1.0, jnp.float32(W)).astype(jnp.int32) + 1
    yh = jnp.clip(y0, -1.0, jnp.float32(H)).astype(jnp.int32) + 1

    lane = lax.broadcasted_iota(jnp.int32, (rows, 64), 1)
    hh = lane // 8
    qq = (lane % 8) // 4
    base = qq * _QSTEP + yh * _YSTEP + xh * _XSTEP + hh
    r00 = base
    r01 = base + _XSTEP
    r10 = base + _YSTEP
    r11 = base + _YSTEP + _XSTEP

    one = jnp.float32(1.0)
    w00 = a * (one - fy) * (one - fx)
    w01 = a * (one - fy) * fx
    w10 = a * fy * (one - fx)
    w11 = a * fy * fx

    i_glob = pl.program_id(0) * rows + lax.broadcasted_iota(
        jnp.int32, (rows, 64), 0)
    valid = i_glob < NQ
    idx = jnp.concatenate([r00, r01, r10, r11], axis=1)
    wts = jnp.concatenate([w00, w01, w10, w11], axis=1)
    vmask = jnp.concatenate([valid] * 4, axis=1)
    idx_ref[...] = jnp.where(vmask, idx, 0)
    w_ref[...] = jnp.where(vmask, wts, jnp.float32(0.0))


def _sc_body(tbl_hbm, idx_hbm, w_hbm, out_hbm, idx_v, w_v, g_v, out_v, sem):
    wid = lax.axis_index("s") * 2 + lax.axis_index("c")

    def chunk(j, _):
        i0 = wid * IPW + j * CI
        base = i0 * 256
        n = CI * 256
        pltpu.sync_copy(idx_hbm.at[pl.ds(base, n)], idx_v)
        pltpu.sync_copy(w_hbm.at[pl.ds(base, n)], w_v)
        pltpu.async_copy(tbl_hbm.at[idx_v], g_v, sem).wait()

        def row(t, _):
            il = t // 16
            rem = t % 16
            qc = rem // 8
            hc = rem % 8
            mb = il * 256 + hc * 8 + qc * 4
            acc0 = jnp.zeros((16,), jnp.float32)
            acc1 = jnp.zeros((16,), jnp.float32)
            for c in range(4):
                for p in range(4):
                    m = mb + c * 64 + p
                    wv = w_v[m]
                    acc0 = acc0 + wv * g_v[m, pl.ds(0, 16)]
                    acc1 = acc1 + wv * g_v[m, pl.ds(16, 16)]
            o = ((qc * CI + il) * NH + hc) * DH
            out_v[pl.ds(o, 16)] = acc0
            out_v[pl.ds(o + 16, 16)] = acc1
            return 0

        lax.fori_loop(0, CI * 2 * NH, row, 0)
        ob = CI * NH * DH
        pltpu.sync_copy(out_v.at[pl.ds(0, ob)],
                        out_hbm.at[0, pl.ds(i0 * NH * DH, ob)])
        pltpu.sync_copy(out_v.at[pl.ds(ob, ob)],
                        out_hbm.at[1, pl.ds(i0 * NH * DH, ob)])
        return 0

    lax.fori_loop(0, NCHUNK, chunk, 0)


def _out_body(s_ref, q_ref, w_ref, b_ref, o_ref):
    s = s_ref[...]
    m = (s[0] + s[1]) * jnp.float32(0.5)
    o_ref[...] = (jnp.dot(m, w_ref[...], preferred_element_type=jnp.float32)
                  + b_ref[...] + q_ref[...])


def kernel(query, value, reference_points, spatial_shapes, level_start_index,
           W_off, b_off, W_attn, b_attn, W_val, b_val, W_out, b_out):
    del spatial_shapes, level_start_index
    f32 = jnp.float32
    q2 = query.reshape(NQ, D)
    v2 = value.reshape(QQ * NQ, D)
    refp = reference_points.reshape(NQ, 2)

    vproj = pl.pallas_call(
        _vproj_body,
        grid=(100,),
        in_specs=[
            pl.BlockSpec((200, D), lambda i: (i, 0)),
            pl.BlockSpec((D, D), lambda i: (0, 0)),
            pl.BlockSpec((1, D), lambda i: (0, 0)),
        ],
        out_specs=pl.BlockSpec((200, D), lambda i: (i, 0)),
        out_shape=jax.ShapeDtypeStruct((QQ * NQ, D), f32),
    )(v2, W_val.T, b_val.reshape(1, D))

    vmap = vproj.reshape(QQ, H, W, NH, DH)
    vmap = jnp.pad(vmap, ((0, 0), (1, 1), (1, 3), (0, 0), (0, 0)))
    table = vmap.reshape(RT, DH)

    wo = W_off.reshape(NH, QQ, P, 2, 2 * D)
    wa = W_attn.reshape(NH * QQ * P, 2 * D)
    bo = b_off.reshape(NH, QQ, P, 2)
    wx = wo[..., 0, :].reshape(64, 2 * D)
    wy = wo[..., 1, :].reshape(64, 2 * D)
    wxv, wxq = wx[:, :D].T, wx[:, D:].T
    wyv, wyq = wy[:, :D].T, wy[:, D:].T
    wav, waq = wa[:, :D].T, wa[:, D:].T
    bx = bo[..., 0].reshape(1, 64)
    by = bo[..., 1].reshape(1, 64)
    ba = b_attn.reshape(1, 64)
    gmat = jnp.kron(jnp.eye(16, dtype=f32), jnp.ones((4, 4), f32))

    rows = 32
    v0p = jnp.pad(v2[:NQ], ((0, NIP - NQ), (0, 0)))
    qp = jnp.pad(q2, ((0, NIP - NQ), (0, 0)))
    refpp = jnp.pad(refp, ((0, NIP - NQ), (0, 0)))
    idx_flat, w_flat = pl.pallas_call(
        functools.partial(_prep_body, rows=rows),
        grid=(NIP // rows,),
        in_specs=[
            pl.BlockSpec((rows, D), lambda i: (i, 0)),
            pl.BlockSpec((rows, D), lambda i: (i, 0)),
            pl.BlockSpec((rows, 2), lambda i: (i, 0)),
        ] + [pl.BlockSpec((D, 64), lambda i: (0, 0))] * 6
        + [pl.BlockSpec((1, 64), lambda i: (0, 0))] * 3
        + [pl.BlockSpec((64, 64), lambda i: (0, 0))],
        out_specs=[
            pl.BlockSpec((rows, 256), lambda i: (i, 0)),
            pl.BlockSpec((rows, 256), lambda i: (i, 0)),
        ],
        out_shape=[
            jax.ShapeDtypeStruct((NIP, 256), jnp.int32),
            jax.ShapeDtypeStruct((NIP, 256), f32),
        ],
    )(v0p, qp, refpp, wxv, wxq, wyv, wyq, wav, waq, bx, by, ba, gmat)

    mesh = plsc.VectorSubcoreMesh(core_axis_name="c", subcore_axis_name="s")
    sampled = pl.kernel(
        _sc_body,
        out_type=jax.ShapeDtypeStruct((QQ, NIP * NH * DH), f32),
        mesh=mesh,
        scratch_types=[
            pltpu.VMEM((CI * 256,), jnp.int32),
            pltpu.VMEM((CI * 256,), f32),
            pltpu.VMEM((CI * 256, DH), f32),
            pltpu.VMEM((QQ * CI * NH * DH,), f32),
            pltpu.SemaphoreType.DMA,
        ],
    )(table, idx_flat.reshape(-1), w_flat.reshape(-1))

    samp = sampled.reshape(QQ, NIP, D)
    out = pl.pallas_call(
        _out_body,
        grid=(125,),
        in_specs=[
            pl.BlockSpec((QQ, 80, D), lambda i: (0, i, 0)),
            pl.BlockSpec((80, D), lambda i: (i, 0)),
            pl.BlockSpec((D, D), lambda i: (0, 0)),
            pl.BlockSpec((1, D), lambda i: (0, 0)),
        ],
        out_specs=pl.BlockSpec((80, D), lambda i: (i, 0)),
        out_shape=jax.ShapeDtypeStruct((NQ, D), f32),
    )(samp, q2, W_out.T, b_out.reshape(1, D))

    return out.reshape(1, NQ, D)


# speedup vs baseline: 26.6172x; 26.6172x over previous
import functools

import jax
import jax.numpy as jnp
from jax import lax
from jax.experimental import pallas as pl
from jax.experimental.pallas import tpu as pltpu
from jax.experimental.pallas import tpu_sc as plsc

NQ = 10000
D = 256
NH = 8
DH = 32
P = 4
QQ = 2
H = 100
W = 100
HP = 102
WP = 104
NW = 32
CI = 4
IPW = 320
NIP = NW * IPW
NCHUNK = IPW // CI
RT = QQ * HP * WP * NH

_XSTEP = NH
_YSTEP = WP * NH
_QSTEP = HP * WP * NH


def _vproj_body(v_ref, w_ref, b_ref, o_ref):
    o_ref[...] = jnp.dot(v_ref[...], w_ref[...],
                         preferred_element_type=jnp.float32) + b_ref[...]


def _prep_body(v0_ref, q_ref, ref_ref, wxv, wxq, wyv, wyq, wav, waq,
               bx_ref, by_ref, ba_ref, g_ref, idx_ref, w_ref, *, rows):
    v0 = v0_ref[...]
    q = q_ref[...]
    dot = functools.partial(jnp.dot, preferred_element_type=jnp.float32)
    offx = dot(v0, wxv[...]) + dot(q, wxq[...]) + bx_ref[...]
    offy = dot(v0, wyv[...]) + dot(q, wyq[...]) + by_ref[...]
    logit = dot(v0, wav[...]) + dot(q, waq[...]) + ba_ref[...]
    logit = logit - jnp.max(logit, axis=1, keepdims=True)
    e = jnp.exp(logit)
    a = e / dot(e, g_ref[...])

    refb = ref_ref[...]
    px = (refb[:, 0:1] + offx / jnp.float32(W)) * jnp.float32(W) - jnp.float32(0.5)
    py = (refb[:, 1:2] + offy / jnp.float32(H)) * jnp.float32(H) - jnp.float32(0.5)
    x0 = jnp.floor(px)
    y0 = jnp.floor(py)
    fx = px - x0
    fy = py - y0
    xh = jnp.clip(x0, -1.0, jnp.float32(W)).astype(jnp.int32) + 1
    yh = jnp.clip(y0, -1.0, jnp.float32(H)).astype(jnp.int32) + 1

    lane = lax.broadcasted_iota(jnp.int32, (rows, 64), 1)
    hh = lane // 8
    qq = (lane % 8) // 4
    base = qq * _QSTEP + yh * _YSTEP + xh * _XSTEP + hh
    r00 = base
    r01 = base + _XSTEP
    r10 = base + _YSTEP
    r11 = base + _YSTEP + _XSTEP

    one = jnp.float32(1.0)
    w00 = a * (one - fy) * (one - fx)
    w01 = a * (one - fy) * fx
    w10 = a * fy * (one - fx)
    w11 = a * fy * fx

    i_glob = pl.program_id(0) * rows + lax.broadcasted_iota(
        jnp.int32, (rows, 64), 0)
    valid = i_glob < NQ
    zi = jnp.int32(0)
    zf = jnp.float32(0.0)
    idx_ref[...] = jnp.concatenate(
        [jnp.where(valid, r, zi) for r in (r00, r01, r10, r11)], axis=1)
    w_ref[...] = jnp.concatenate(
        [jnp.where(valid, w, zf) for w in (w00, w01, w10, w11)], axis=1)


def _sc_body(tbl_hbm, idx_hbm, w_hbm, out_hbm, idx_v, w_v, g_v, out_v, sem):
    wid = lax.axis_index("s") * 2 + lax.axis_index("c")

    def chunk(j, _):
        i0 = wid * IPW + j * CI
        base = i0 * 256
        n = CI * 256
        pltpu.sync_copy(idx_hbm.at[pl.ds(base, n)], idx_v)
        pltpu.sync_copy(w_hbm.at[pl.ds(base, n)], w_v)
        pltpu.async_copy(tbl_hbm.at[idx_v], g_v, sem).wait()

        def row(t, _):
            il = t // 16
            rem = t % 16
            qc = rem // 8
            hc = rem % 8
            mb = il * 256 + hc * 8 + qc * 4
            acc0 = jnp.zeros((16,), jnp.float32)
            acc1 = jnp.zeros((16,), jnp.float32)
            for c in range(4):
                for p in range(4):
                    m = mb + c * 64 + p
                    wv = plsc.load_gather(w_v, [jnp.broadcast_to(m, (16,))])
                    acc0 = acc0 + wv * g_v[m, pl.ds(0, 16)]
                    acc1 = acc1 + wv * g_v[m, pl.ds(16, 16)]
            o = ((qc * CI + il) * NH + hc) * DH
            out_v[pl.ds(o, 16)] = acc0
            out_v[pl.ds(o + 16, 16)] = acc1
            return 0

        lax.fori_loop(0, CI * 2 * NH, row, 0)
        ob = CI * NH * DH
        pltpu.sync_copy(out_v.at[pl.ds(0, ob)],
                        out_hbm.at[pl.ds(i0 * NH * DH, ob)])
        pltpu.sync_copy(out_v.at[pl.ds(ob, ob)],
                        out_hbm.at[pl.ds(NIP * NH * DH + i0 * NH * DH, ob)])
        return 0

    lax.fori_loop(0, NCHUNK, chunk, 0)


def _out_body(s_ref, q_ref, w_ref, b_ref, o_ref):
    s = s_ref[...]
    m = (s[0] + s[1]) * jnp.float32(0.5)
    o_ref[...] = (jnp.dot(m, w_ref[...], preferred_element_type=jnp.float32)
                  + b_ref[...] + q_ref[...])


def kernel(query, value, reference_points, spatial_shapes, level_start_index,
           W_off, b_off, W_attn, b_attn, W_val, b_val, W_out, b_out):
    del spatial_shapes, level_start_index
    f32 = jnp.float32
    q2 = query.reshape(NQ, D)
    v2 = value.reshape(QQ * NQ, D)
    refp = reference_points.reshape(NQ, 2)

    vproj = pl.pallas_call(
        _vproj_body,
        grid=(100,),
        in_specs=[
            pl.BlockSpec((200, D), lambda i: (i, 0)),
            pl.BlockSpec((D, D), lambda i: (0, 0)),
            pl.BlockSpec((1, D), lambda i: (0, 0)),
        ],
        out_specs=pl.BlockSpec((200, D), lambda i: (i, 0)),
        out_shape=jax.ShapeDtypeStruct((QQ * NQ, D), f32),
    )(v2, W_val.T, b_val.reshape(1, D))

    vmap = vproj.reshape(QQ, H, W, NH, DH)
    vmap = jnp.pad(vmap, ((0, 0), (1, 1), (1, 3), (0, 0), (0, 0)))
    table = vmap.reshape(RT, DH)

    wo = W_off.reshape(NH, QQ, P, 2, 2 * D)
    wa = W_attn.reshape(NH * QQ * P, 2 * D)
    bo = b_off.reshape(NH, QQ, P, 2)
    wx = wo[..., 0, :].reshape(64, 2 * D)
    wy = wo[..., 1, :].reshape(64, 2 * D)
    wxv, wxq = wx[:, :D].T, wx[:, D:].T
    wyv, wyq = wy[:, :D].T, wy[:, D:].T
    wav, waq = wa[:, :D].T, wa[:, D:].T
    bx = bo[..., 0].reshape(1, 64)
    by = bo[..., 1].reshape(1, 64)
    ba = b_attn.reshape(1, 64)
    gmat = jnp.kron(jnp.eye(16, dtype=f32), jnp.ones((4, 4), f32))

    rows = 32
    v0p = jnp.pad(v2[:NQ], ((0, NIP - NQ), (0, 0)))
    qp = jnp.pad(q2, ((0, NIP - NQ), (0, 0)))
    refpp = jnp.pad(refp, ((0, NIP - NQ), (0, 0)))
    idx_flat, w_flat = pl.pallas_call(
        functools.partial(_prep_body, rows=rows),
        grid=(NIP // rows,),
        in_specs=[
            pl.BlockSpec((rows, D), lambda i: (i, 0)),
            pl.BlockSpec((rows, D), lambda i: (i, 0)),
            pl.BlockSpec((rows, 2), lambda i: (i, 0)),
        ] + [pl.BlockSpec((D, 64), lambda i: (0, 0))] * 6
        + [pl.BlockSpec((1, 64), lambda i: (0, 0))] * 3
        + [pl.BlockSpec((64, 64), lambda i: (0, 0))],
        out_specs=[
            pl.BlockSpec((rows, 256), lambda i: (i, 0)),
            pl.BlockSpec((rows, 256), lambda i: (i, 0)),
        ],
        out_shape=[
            jax.ShapeDtypeStruct((NIP, 256), jnp.int32),
            jax.ShapeDtypeStruct((NIP, 256), f32),
        ],
    )(v0p, qp, refpp, wxv, wxq, wyv, wyq, wav, waq, bx, by, ba, gmat)

    mesh = plsc.VectorSubcoreMesh(core_axis_name="c", subcore_axis_name="s")
    sampled = pl.kernel(
        _sc_body,
        out_type=jax.ShapeDtypeStruct((QQ * NIP * NH * DH,), f32),
        mesh=mesh,
        compiler_params=pltpu.CompilerParams(use_tc_tiling_on_sc=False,
                                             needs_layout_passes=False),
        scratch_types=[
            pltpu.VMEM((CI * 256,), jnp.int32),
            pltpu.VMEM((CI * 256,), f32),
            pltpu.VMEM((CI * 256, DH), f32),
            pltpu.VMEM((QQ * CI * NH * DH,), f32),
            pltpu.SemaphoreType.DMA,
        ],
    )(table, idx_flat.reshape(-1), w_flat.reshape(-1))

    samp = sampled.reshape(QQ, NIP, D)
    out = pl.pallas_call(
        _out_body,
        grid=(125,),
        in_specs=[
            pl.BlockSpec((QQ, 80, D), lambda i: (0, i, 0)),
            pl.BlockSpec((80, D), lambda i: (i, 0)),
            pl.BlockSpec((D, D), lambda i: (0, 0)),
            pl.BlockSpec((1, D), lambda i: (0, 0)),
        ],
        out_specs=pl.BlockSpec((80, D), lambda i: (i, 0)),
        out_shape=jax.ShapeDtypeStruct((NQ, D), f32),
    )(samp, q2, W_out.T, b_out.reshape(1, D))

    return out.reshape(1, NQ, D)


# speedup vs baseline: 37.7507x; 1.4183x over previous
import functools

import numpy as np
import jax
import jax.numpy as jnp
from jax import lax
from jax.experimental import pallas as pl
from jax.experimental.pallas import tpu as pltpu
from jax.experimental.pallas import tpu_sc as plsc

NQ = 10000
D = 256
NH = 8
DH = 32
P = 4
QQ = 2
H = 100
W = 100
HP = 102
WP = 104
NW = 32
CI = 4
IPW = 320
NIP = NW * IPW
NCHUNK = IPW // CI
RT = QQ * HP * WP * NH

_SIGMA = np.concatenate(
    [h * DH + np.concatenate([np.arange(16) * 2, np.arange(16) * 2 + 1])
     for h in range(NH)])

_XSTEP = NH
_YSTEP = WP * NH
_QSTEP = HP * WP * NH


def _vproj_body(v_ref, w_ref, b_ref, o_ref):
    o_ref[...] = (jnp.dot(v_ref[...], w_ref[...],
                          preferred_element_type=jnp.float32)
                  + b_ref[...]).astype(jnp.bfloat16)


def _prep_body(v0_ref, q_ref, ref_ref, wxv, wxq, wyv, wyq, wav, waq,
               bx_ref, by_ref, ba_ref, g_ref, idx_ref, w_ref, *, rows):
    v0 = v0_ref[...]
    q = q_ref[...]
    dot = functools.partial(jnp.dot, preferred_element_type=jnp.float32)
    offx = dot(v0, wxv[...]) + dot(q, wxq[...]) + bx_ref[...]
    offy = dot(v0, wyv[...]) + dot(q, wyq[...]) + by_ref[...]
    logit = dot(v0, wav[...]) + dot(q, waq[...]) + ba_ref[...]
    logit = logit - jnp.max(logit, axis=1, keepdims=True)
    e = jnp.exp(logit)
    a = e / dot(e, g_ref[...])

    refb = ref_ref[...]
    px = (refb[:, 0:1] + offx / jnp.float32(W)) * jnp.float32(W) - jnp.float32(0.5)
    py = (refb[:, 1:2] + offy / jnp.float32(H)) * jnp.float32(H) - jnp.float32(0.5)
    x0 = jnp.floor(px)
    y0 = jnp.floor(py)
    fx = px - x0
    fy = py - y0
    xh = jnp.clip(x0, -1.0, jnp.float32(W)).astype(jnp.int32) + 1
    yh = jnp.clip(y0, -1.0, jnp.float32(H)).astype(jnp.int32) + 1

    lane = lax.broadcasted_iota(jnp.int32, (rows, 64), 1)
    hh = lane // 8
    qq = (lane % 8) // 4
    base = qq * _QSTEP + yh * _YSTEP + xh * _XSTEP + hh
    r00 = base
    r01 = base + _XSTEP
    r10 = base + _YSTEP
    r11 = base + _YSTEP + _XSTEP

    one = jnp.float32(1.0)
    w00 = a * (one - fy) * (one - fx)
    w01 = a * (one - fy) * fx
    w10 = a * fy * (one - fx)
    w11 = a * fy * fx

    i_glob = pl.program_id(0) * rows + lax.broadcasted_iota(
        jnp.int32, (rows, 64), 0)
    valid = i_glob < NQ
    zi = jnp.int32(0)
    zf = jnp.float32(0.0)
    idx_ref[...] = jnp.concatenate(
        [jnp.where(valid, r, zi) for r in (r00, r01, r10, r11)], axis=1)
    w_ref[...] = jnp.concatenate(
        [jnp.where(valid, w, zf) for w in (w00, w01, w10, w11)], axis=1)


def _sc_body(tbl_hbm, idx_hbm, w_hbm, out_hbm, idx_v, w_v, g_v, out_v, sem):
    wid = lax.axis_index("s") * 2 + lax.axis_index("c")

    def chunk(j, _):
        i0 = wid * IPW + j * CI
        base = i0 * 256
        n = CI * 256
        pltpu.sync_copy(idx_hbm.at[pl.ds(base, n)], idx_v)
        pltpu.sync_copy(w_hbm.at[pl.ds(base, n)], w_v)
        pltpu.async_copy(tbl_hbm.at[idx_v], g_v, sem).wait()

        def row(t, _):
            il = t // 16
            rem = t % 16
            qc = rem // 8
            hc = rem % 8
            mb = il * 256 + hc * 8 + qc * 4
            acc0 = jnp.zeros((16,), jnp.float32)
            acc1 = jnp.zeros((16,), jnp.float32)
            for c in range(4):
                for p in range(4):
                    m = mb + c * 64 + p
                    wv = plsc.load_gather(w_v, [jnp.broadcast_to(m, (16,))])
                    ga, gb = plsc.unpack(g_v[m, pl.ds(0, 32)],
                                         format=plsc.PackFormat.INTERLEAVED)
                    acc0 = acc0 + wv * ga
                    acc1 = acc1 + wv * gb
            o = ((qc * CI + il) * NH + hc) * DH
            out_v[pl.ds(o, 16)] = acc0
            out_v[pl.ds(o + 16, 16)] = acc1
            return 0

        lax.fori_loop(0, CI * 2 * NH, row, 0)
        ob = CI * NH * DH
        pltpu.sync_copy(out_v.at[pl.ds(0, ob)],
                        out_hbm.at[pl.ds(i0 * NH * DH, ob)])
        pltpu.sync_copy(out_v.at[pl.ds(ob, ob)],
                        out_hbm.at[pl.ds(NIP * NH * DH + i0 * NH * DH, ob)])
        return 0

    lax.fori_loop(0, NCHUNK, chunk, 0)


def _out_body(s_ref, q_ref, w_ref, b_ref, o_ref):
    s = s_ref[...]
    m = (s[0] + s[1]) * jnp.float32(0.5)
    o_ref[...] = (jnp.dot(m, w_ref[...], preferred_element_type=jnp.float32)
                  + b_ref[...] + q_ref[...])


def kernel(query, value, reference_points, spatial_shapes, level_start_index,
           W_off, b_off, W_attn, b_attn, W_val, b_val, W_out, b_out):
    del spatial_shapes, level_start_index
    f32 = jnp.float32
    q2 = query.reshape(NQ, D)
    v2 = value.reshape(QQ * NQ, D)
    refp = reference_points.reshape(NQ, 2)

    vproj = pl.pallas_call(
        _vproj_body,
        grid=(100,),
        in_specs=[
            pl.BlockSpec((200, D), lambda i: (i, 0)),
            pl.BlockSpec((D, D), lambda i: (0, 0)),
            pl.BlockSpec((1, D), lambda i: (0, 0)),
        ],
        out_specs=pl.BlockSpec((200, D), lambda i: (i, 0)),
        out_shape=jax.ShapeDtypeStruct((QQ * NQ, D), jnp.bfloat16),
    )(v2, W_val.T, b_val.reshape(1, D))

    vmap = vproj.reshape(QQ, H, W, NH, DH)
    vmap = jnp.pad(vmap, ((0, 0), (1, 1), (1, 3), (0, 0), (0, 0)))
    table = vmap.reshape(RT, DH)

    wo = W_off.reshape(NH, QQ, P, 2, 2 * D)
    wa = W_attn.reshape(NH * QQ * P, 2 * D)
    bo = b_off.reshape(NH, QQ, P, 2)
    wx = wo[..., 0, :].reshape(64, 2 * D)
    wy = wo[..., 1, :].reshape(64, 2 * D)
    wxv, wxq = wx[:, :D].T, wx[:, D:].T
    wyv, wyq = wy[:, :D].T, wy[:, D:].T
    wav, waq = wa[:, :D].T, wa[:, D:].T
    bx = bo[..., 0].reshape(1, 64)
    by = bo[..., 1].reshape(1, 64)
    ba = b_attn.reshape(1, 64)
    gmat = jnp.kron(jnp.eye(16, dtype=f32), jnp.ones((4, 4), f32))

    rows = 32
    v0p = jnp.pad(v2[:NQ], ((0, NIP - NQ), (0, 0)))
    qp = jnp.pad(q2, ((0, NIP - NQ), (0, 0)))
    refpp = jnp.pad(refp, ((0, NIP - NQ), (0, 0)))
    idx_flat, w_flat = pl.pallas_call(
        functools.partial(_prep_body, rows=rows),
        grid=(NIP // rows,),
        in_specs=[
            pl.BlockSpec((rows, D), lambda i: (i, 0)),
            pl.BlockSpec((rows, D), lambda i: (i, 0)),
            pl.BlockSpec((rows, 2), lambda i: (i, 0)),
        ] + [pl.BlockSpec((D, 64), lambda i: (0, 0))] * 6
        + [pl.BlockSpec((1, 64), lambda i: (0, 0))] * 3
        + [pl.BlockSpec((64, 64), lambda i: (0, 0))],
        out_specs=[
            pl.BlockSpec((rows, 256), lambda i: (i, 0)),
            pl.BlockSpec((rows, 256), lambda i: (i, 0)),
        ],
        out_shape=[
            jax.ShapeDtypeStruct((NIP, 256), jnp.int32),
            jax.ShapeDtypeStruct((NIP, 256), f32),
        ],
    )(v0p, qp, refpp, wxv, wxq, wyv, wyq, wav, waq, bx, by, ba, gmat)

    mesh = plsc.VectorSubcoreMesh(core_axis_name="c", subcore_axis_name="s")
    sampled = pl.kernel(
        _sc_body,
        out_type=jax.ShapeDtypeStruct((QQ * NIP * NH * DH,), f32),
        mesh=mesh,
        compiler_params=pltpu.CompilerParams(use_tc_tiling_on_sc=False,
                                             needs_layout_passes=False),
        scratch_types=[
            pltpu.VMEM((CI * 256,), jnp.int32),
            pltpu.VMEM((CI * 256,), f32),
            pltpu.VMEM((CI * 256, DH), jnp.bfloat16),
            pltpu.VMEM((QQ * CI * NH * DH,), f32),
            pltpu.SemaphoreType.DMA,
        ],
    )(table, idx_flat.reshape(-1), w_flat.reshape(-1))

    samp = sampled.reshape(QQ, NIP, D)
    out = pl.pallas_call(
        _out_body,
        grid=(125,),
        in_specs=[
            pl.BlockSpec((QQ, 80, D), lambda i: (0, i, 0)),
            pl.BlockSpec((80, D), lambda i: (i, 0)),
            pl.BlockSpec((D, D), lambda i: (0, 0)),
            pl.BlockSpec((1, D), lambda i: (0, 0)),
        ],
        out_specs=pl.BlockSpec((80, D), lambda i: (i, 0)),
        out_shape=jax.ShapeDtypeStruct((NQ, D), f32),
    )(samp, q2, W_out.T[_SIGMA], b_out.reshape(1, D))

    return out.reshape(1, NQ, D)


# speedup vs baseline: 44.9480x; 1.1907x over previous
import functools

import numpy as np
import jax
import jax.numpy as jnp
from jax import lax
from jax.experimental import pallas as pl
from jax.experimental.pallas import tpu as pltpu
from jax.experimental.pallas import tpu_sc as plsc

NQ = 10000
D = 256
NH = 8
DH = 32
P = 4
QQ = 2
H = 100
W = 100
HP = 102
WP = 104
NW = 32
CI = 4
IPW = 320
NIP = NW * IPW
NCHUNK = IPW // CI
RT = QQ * HP * WP * NH

_SIGMA = np.concatenate(
    [h * DH + np.concatenate([np.arange(16) * 2, np.arange(16) * 2 + 1])
     for h in range(NH)])

_XSTEP = NH
_YSTEP = WP * NH
_QSTEP = HP * WP * NH


def _vproj_body(v_ref, w_ref, b_ref, o_ref):
    o_ref[...] = (jnp.dot(v_ref[...], w_ref[...],
                          preferred_element_type=jnp.float32)
                  + b_ref[...]).astype(jnp.bfloat16)


def _prep_body(v0_ref, q_ref, ref_ref, wxv, wxq, wyv, wyq, wav, waq,
               bx_ref, by_ref, ba_ref, g_ref, idx_ref, w_ref, *, rows):
    v0 = v0_ref[...]
    q = q_ref[...]
    dot = functools.partial(jnp.dot, preferred_element_type=jnp.float32)
    offx = dot(v0, wxv[...]) + dot(q, wxq[...]) + bx_ref[...]
    offy = dot(v0, wyv[...]) + dot(q, wyq[...]) + by_ref[...]
    logit = dot(v0, wav[...]) + dot(q, waq[...]) + ba_ref[...]
    logit = logit - jnp.max(logit, axis=1, keepdims=True)
    e = jnp.exp(logit)
    a = e / dot(e, g_ref[...])

    refb = ref_ref[...]
    px = (refb[:, 0:1] + offx / jnp.float32(W)) * jnp.float32(W) - jnp.float32(0.5)
    py = (refb[:, 1:2] + offy / jnp.float32(H)) * jnp.float32(H) - jnp.float32(0.5)
    x0 = jnp.floor(px)
    y0 = jnp.floor(py)
    fx = px - x0
    fy = py - y0
    xh = jnp.clip(x0, -1.0, jnp.float32(W)).astype(jnp.int32) + 1
    yh = jnp.clip(y0, -1.0, jnp.float32(H)).astype(jnp.int32) + 1

    lane = lax.broadcasted_iota(jnp.int32, (rows, 64), 1)
    hh = lane // 8
    qq = (lane % 8) // 4
    base = qq * _QSTEP + yh * _YSTEP + xh * _XSTEP + hh
    r00 = base
    r01 = base + _XSTEP
    r10 = base + _YSTEP
    r11 = base + _YSTEP + _XSTEP

    one = jnp.float32(1.0)
    w00 = a * (one - fy) * (one - fx)
    w01 = a * (one - fy) * fx
    w10 = a * fy * (one - fx)
    w11 = a * fy * fx

    i_glob = pl.program_id(0) * rows + lax.broadcasted_iota(
        jnp.int32, (rows, 64), 0)
    valid = i_glob < NQ
    zi = jnp.int32(0)
    zf = jnp.float32(0.0)
    idx_ref[...] = jnp.concatenate(
        [jnp.where(valid, r, zi) for r in (r00, r01, r10, r11)], axis=1)
    w_ref[...] = jnp.concatenate(
        [jnp.where(valid, w, zf) for w in (w00, w01, w10, w11)], axis=1)


_N = CI * 256
_OB = CI * NH * DH


def _compute_chunk(w_v, g_v, out_v):
    def row(t, _):
        il = t // 16
        rem = t % 16
        qc = rem // 8
        hc = rem % 8
        mb = il * 256 + hc * 8 + qc * 4
        acc0 = jnp.zeros((16,), jnp.float32)
        acc1 = jnp.zeros((16,), jnp.float32)
        for c in range(4):
            for p in range(4):
                m = mb + c * 64 + p
                wv = plsc.load_gather(w_v, [jnp.broadcast_to(m, (16,))])
                ga, gb = plsc.unpack(g_v[m, pl.ds(0, 32)],
                                     format=plsc.PackFormat.INTERLEAVED)
                acc0 = acc0 + wv * ga
                acc1 = acc1 + wv * gb
        o = ((qc * CI + il) * NH + hc) * DH
        out_v[pl.ds(o, 16)] = acc0
        out_v[pl.ds(o + 16, 16)] = acc1
        return 0

    lax.fori_loop(0, CI * 2 * NH, row, 0)


def _sc_body(tbl_hbm, idx_hbm, w_hbm, out_hbm,
             i0v, i1v, w0v, w1v, g0v, g1v, o0v, o1v,
             sf0, sf1, sg0, sg1, so0, so1):
    wid = lax.axis_index("s") * 2 + lax.axis_index("c")
    idxv = (i0v, i1v)
    wv = (w0v, w1v)
    gv = (g0v, g1v)
    ov = (o0v, o1v)
    sf = (sf0, sf1)
    sg = (sg0, sg1)
    so = (so0, so1)

    def base(j):
        return (wid * IPW + jnp.minimum(j, NCHUNK - 1) * CI) * 256

    def fetch_start(j, par):
        pltpu.async_copy(idx_hbm.at[pl.ds(base(j), _N)], idxv[par], sf[par])
        pltpu.async_copy(w_hbm.at[pl.ds(base(j), _N)], wv[par], sf[par])

    def fetch_wait(j, par):
        pltpu.make_async_copy(idx_hbm.at[pl.ds(base(j), _N)], idxv[par],
                              sf[par]).wait()
        pltpu.make_async_copy(w_hbm.at[pl.ds(base(j), _N)], wv[par],
                              sf[par]).wait()

    def gather_start(par):
        pltpu.async_copy(tbl_hbm.at[idxv[par]], gv[par], sg[par])

    def gather_wait(par):
        pltpu.make_async_copy(tbl_hbm.at[idxv[par]], gv[par], sg[par]).wait()

    def out_start(j, par):
        ob = base(j)
        pltpu.async_copy(ov[par].at[pl.ds(0, _OB)],
                         out_hbm.at[pl.ds(ob, _OB)], so[par])
        pltpu.async_copy(ov[par].at[pl.ds(_OB, _OB)],
                         out_hbm.at[pl.ds(NIP * NH * DH + ob, _OB)], so[par])

    def out_wait(j, par):
        ob = base(j)
        pltpu.make_async_copy(ov[par].at[pl.ds(0, _OB)],
                              out_hbm.at[pl.ds(ob, _OB)], so[par]).wait()
        pltpu.make_async_copy(ov[par].at[pl.ds(_OB, _OB)],
                              out_hbm.at[pl.ds(NIP * NH * DH + ob, _OB)],
                              so[par]).wait()

    fetch_start(0, 0)
    fetch_wait(0, 0)
    gather_start(0)
    fetch_start(1, 1)

    def step(cc, _):
        for par in range(2):
            j = cc * 2 + par
            gather_wait(par)
            fetch_wait(j + 1, 1 - par)
            gather_start(1 - par)

            @pl.when(cc > 0)
            def _():
                out_wait(j - 2, par)

            _compute_chunk(wv[par], gv[par], ov[par])
            out_start(j, par)
            fetch_start(j + 2, par)
        return 0

    lax.fori_loop(0, NCHUNK // 2, step, 0)

    gather_wait(0)
    fetch_wait(NCHUNK, 1)
    out_wait(NCHUNK - 2, 0)
    out_wait(NCHUNK - 1, 1)


def _out_body(s_ref, q_ref, w_ref, b_ref, o_ref):
    s = s_ref[...]
    m = (s[0] + s[1]) * jnp.float32(0.5)
    o_ref[...] = (jnp.dot(m, w_ref[...], preferred_element_type=jnp.float32)
                  + b_ref[...] + q_ref[...])


def kernel(query, value, reference_points, spatial_shapes, level_start_index,
           W_off, b_off, W_attn, b_attn, W_val, b_val, W_out, b_out):
    del spatial_shapes, level_start_index
    f32 = jnp.float32
    q2 = query.reshape(NQ, D)
    v2 = value.reshape(QQ * NQ, D)
    refp = reference_points.reshape(NQ, 2)

    vproj = pl.pallas_call(
        _vproj_body,
        grid=(100,),
        in_specs=[
            pl.BlockSpec((200, D), lambda i: (i, 0)),
            pl.BlockSpec((D, D), lambda i: (0, 0)),
            pl.BlockSpec((1, D), lambda i: (0, 0)),
        ],
        out_specs=pl.BlockSpec((200, D), lambda i: (i, 0)),
        out_shape=jax.ShapeDtypeStruct((QQ * NQ, D), jnp.bfloat16),
    )(v2, W_val.T, b_val.reshape(1, D))

    vmap = vproj.reshape(QQ, H, W, NH, DH)
    vmap = jnp.pad(vmap, ((0, 0), (1, 1), (1, 3), (0, 0), (0, 0)))
    table = vmap.reshape(RT, DH)

    wo = W_off.reshape(NH, QQ, P, 2, 2 * D)
    wa = W_attn.reshape(NH * QQ * P, 2 * D)
    bo = b_off.reshape(NH, QQ, P, 2)
    wx = wo[..., 0, :].reshape(64, 2 * D)
    wy = wo[..., 1, :].reshape(64, 2 * D)
    wxv, wxq = wx[:, :D].T, wx[:, D:].T
    wyv, wyq = wy[:, :D].T, wy[:, D:].T
    wav, waq = wa[:, :D].T, wa[:, D:].T
    bx = bo[..., 0].reshape(1, 64)
    by = bo[..., 1].reshape(1, 64)
    ba = b_attn.reshape(1, 64)
    gmat = jnp.kron(jnp.eye(16, dtype=f32), jnp.ones((4, 4), f32))

    rows = 32
    v0p = jnp.pad(v2[:NQ], ((0, NIP - NQ), (0, 0)))
    qp = jnp.pad(q2, ((0, NIP - NQ), (0, 0)))
    refpp = jnp.pad(refp, ((0, NIP - NQ), (0, 0)))
    idx_flat, w_flat = pl.pallas_call(
        functools.partial(_prep_body, rows=rows),
        grid=(NIP // rows,),
        in_specs=[
            pl.BlockSpec((rows, D), lambda i: (i, 0)),
            pl.BlockSpec((rows, D), lambda i: (i, 0)),
            pl.BlockSpec((rows, 2), lambda i: (i, 0)),
        ] + [pl.BlockSpec((D, 64), lambda i: (0, 0))] * 6
        + [pl.BlockSpec((1, 64), lambda i: (0, 0))] * 3
        + [pl.BlockSpec((64, 64), lambda i: (0, 0))],
        out_specs=[
            pl.BlockSpec((rows, 256), lambda i: (i, 0)),
            pl.BlockSpec((rows, 256), lambda i: (i, 0)),
        ],
        out_shape=[
            jax.ShapeDtypeStruct((NIP, 256), jnp.int32),
            jax.ShapeDtypeStruct((NIP, 256), f32),
        ],
    )(v0p, qp, refpp, wxv, wxq, wyv, wyq, wav, waq, bx, by, ba, gmat)

    mesh = plsc.VectorSubcoreMesh(core_axis_name="c", subcore_axis_name="s")
    sampled = pl.kernel(
        _sc_body,
        out_type=jax.ShapeDtypeStruct((QQ * NIP * NH * DH,), f32),
        mesh=mesh,
        compiler_params=pltpu.CompilerParams(use_tc_tiling_on_sc=False,
                                             needs_layout_passes=False),
        scratch_types=(
            [pltpu.VMEM((_N,), jnp.int32)] * 2
            + [pltpu.VMEM((_N,), f32)] * 2
            + [pltpu.VMEM((_N, DH), jnp.bfloat16)] * 2
            + [pltpu.VMEM((QQ * _OB,), f32)] * 2
            + [pltpu.SemaphoreType.DMA] * 6
        ),
    )(table, idx_flat.reshape(-1), w_flat.reshape(-1))

    samp = sampled.reshape(QQ, NIP, D)
    out = pl.pallas_call(
        _out_body,
        grid=(125,),
        in_specs=[
            pl.BlockSpec((QQ, 80, D), lambda i: (0, i, 0)),
            pl.BlockSpec((80, D), lambda i: (i, 0)),
            pl.BlockSpec((D, D), lambda i: (0, 0)),
            pl.BlockSpec((1, D), lambda i: (0, 0)),
        ],
        out_specs=pl.BlockSpec((80, D), lambda i: (i, 0)),
        out_shape=jax.ShapeDtypeStruct((NQ, D), f32),
    )(samp, q2, W_out.T[_SIGMA], b_out.reshape(1, D))

    return out.reshape(1, NQ, D)
